# baseline (speedup 1.0000x reference)
import functools

import jax
import jax.numpy as jnp
from jax import lax
from jax.experimental import pallas as pl
from jax.experimental.pallas import tpu as pltpu
from jax.experimental.pallas import tpu_sc as plsc

NC = 2
NS = 16
NW = NC * NS

BATCH = 16384
D = 32
B_PER_W = BATCH // NW
CHUNK = 128
N_CHUNK = B_PER_W // CHUNK
IDX_ROWS = BATCH // CHUNK


def _gather_body(ut_hbm, it_hbm, ui_hbm, ii_hbm, u_hbm, v_hbm,
                 uidx_v, iidx_v, urows_v, irows_v, sem):
    wid = lax.axis_index("s") * NC + lax.axis_index("c")
    row0 = wid * N_CHUNK
    pltpu.sync_copy(ui_hbm.at[pl.ds(row0, N_CHUNK)], uidx_v)
    pltpu.sync_copy(ii_hbm.at[pl.ds(row0, N_CHUNK)], iidx_v)
    copies = []
    for j in range(N_CHUNK):
        copies.append(pltpu.async_copy(
            ut_hbm.at[uidx_v.at[j]], urows_v.at[pl.ds(j * CHUNK, CHUNK)], sem))
        copies.append(pltpu.async_copy(
            it_hbm.at[iidx_v.at[j]], irows_v.at[pl.ds(j * CHUNK, CHUNK)], sem))
    for c in copies:
        c.wait()
    base = wid * B_PER_W
    pltpu.sync_copy(urows_v, u_hbm.at[pl.ds(base, B_PER_W)])
    pltpu.sync_copy(irows_v, v_hbm.at[pl.ds(base, B_PER_W)])


def _sc_gather(user_table, item_table, user_idx, item_idx):
    mesh = plsc.VectorSubcoreMesh(core_axis_name="c", subcore_axis_name="s")
    rows_t = jax.ShapeDtypeStruct((BATCH, D), jnp.float32)
    k = pl.kernel(
        _gather_body,
        out_type=[rows_t, rows_t],
        mesh=mesh,
        compiler_params=pltpu.CompilerParams(use_tc_tiling_on_sc=False),
        scratch_types=[
            pltpu.VMEM((N_CHUNK, CHUNK), jnp.int32),
            pltpu.VMEM((N_CHUNK, CHUNK), jnp.int32),
            pltpu.VMEM((B_PER_W, D), jnp.float32),
            pltpu.VMEM((B_PER_W, D), jnp.float32),
            pltpu.SemaphoreType.DMA,
        ],
    )
    return k(user_table, item_table,
             user_idx.reshape(IDX_ROWS, CHUNK), item_idx.reshape(IDX_ROWS, CHUNK))


BLK = 2048


def _mlp_body(u_ref, v_ref, w1u_ref, w1v_ref, b1_ref, w2_ref, b2_ref,
              wo_ref, bo_ref, o_ref):
    h = u_ref[...] @ w1u_ref[...] + v_ref[...] @ w1v_ref[...] + b1_ref[...]
    h = jnp.maximum(h, 0.0)
    h = jnp.maximum(h @ w2_ref[...] + b2_ref[...], 0.0)
    o_ref[...] = h @ wo_ref[...] + bo_ref[...]


def _tc_mlp(u, v, W1, b1, W2, b2, Wo, bo):
    w1u, w1v = W1[:D], W1[D:]
    grid = (BATCH // BLK,)
    full = lambda shape: pl.BlockSpec(shape, lambda i: (0, 0))
    out = pl.pallas_call(
        _mlp_body,
        grid=grid,
        in_specs=[
            pl.BlockSpec((BLK, D), lambda i: (i, 0)),
            pl.BlockSpec((BLK, D), lambda i: (i, 0)),
            full((D, 64)),
            full((D, 64)),
            full((1, 64)),
            full((64, 32)),
            full((1, 32)),
            full((32, 1)),
            full((1, 1)),
        ],
        out_specs=pl.BlockSpec((BLK, 1), lambda i: (i, 0)),
        out_shape=jax.ShapeDtypeStruct((BATCH, 1), jnp.float32),
    )(u, v, w1u, w1v, b1.reshape(1, 64), W2, b2.reshape(1, 32),
      Wo, bo.reshape(1, 1))
    return out[:, 0]


def kernel(user_indices, item_indices, user_table, item_table,
           W1, b1, W2, b2, Wo, bo):
    u, v = _sc_gather(user_table, item_table, user_indices, item_indices)
    return _tc_mlp(u, v, W1, b1, W2, b2, Wo, bo)
